# Initial kernel scaffold; baseline (speedup 1.0000x reference)
import jax
import jax.numpy as jnp
from jax.experimental import pallas as pl


def kernel(byte_ids, table, gamma, beta):
    raise NotImplementedError("write your pallas kernel here")


# speedup vs baseline: 2.0074x; 2.0074x over previous
import functools

import jax
import jax.numpy as jnp
from jax import lax
from jax.experimental import pallas as pl
from jax.experimental.pallas import tpu as pltpu
from jax.experimental.pallas import tpu_sc as plsc

D_MODEL = 1024
NUM_ROWS = 256
NUM_CORES = 2
NUM_SUBCORES = 16
NUM_WORKERS = NUM_CORES * NUM_SUBCORES
LANES = 16


def _ln_table_body(table_ref, gamma_ref, beta_ref, out_ref):
    x = table_ref[...]
    mean = jnp.mean(x, axis=1, keepdims=True)
    cent = x - mean
    var = jnp.mean(cent * cent, axis=1, keepdims=True)
    inv = lax.rsqrt(var + 1e-5)
    out_ref[...] = cent * inv * gamma_ref[...] + beta_ref[...]


def _normalize_table(table, gamma, beta):
    return pl.pallas_call(
        _ln_table_body,
        out_shape=jax.ShapeDtypeStruct((NUM_ROWS, D_MODEL), jnp.float32),
    )(table, gamma.reshape(1, D_MODEL), beta.reshape(1, D_MODEL))


def _make_sc_gather(total_tokens):
    assert total_tokens % (8 * NUM_WORKERS) == 0
    tokens_per_worker = total_tokens // NUM_WORKERS
    chunk = 32
    n_chunks = tokens_per_worker // chunk
    mesh = plsc.VectorSubcoreMesh(
        core_axis_name="c",
        subcore_axis_name="s",
        num_cores=NUM_CORES,
        num_subcores=NUM_SUBCORES,
    )

    @functools.partial(
        pl.kernel,
        out_type=jax.ShapeDtypeStruct((total_tokens, D_MODEL), jnp.float32),
        mesh=mesh,
        scratch_types=[
            pltpu.VMEM((tokens_per_worker,), jnp.int32),
            pltpu.VMEM((chunk, D_MODEL), jnp.float32),
            pltpu.SemaphoreType.DMA,
        ],
    )
    def sc_gather(tab_hbm, idx_hbm, out_hbm, idx_v, rows_v, sem):
        wid = lax.axis_index("s") * NUM_CORES + lax.axis_index("c")
        base = wid * tokens_per_worker
        pltpu.sync_copy(idx_hbm.at[pl.ds(base, tokens_per_worker)], idx_v)

        def clamp_body(i, carry):
            v = idx_v[pl.ds(i * LANES, LANES)]
            idx_v[pl.ds(i * LANES, LANES)] = jnp.clip(v, 0, NUM_ROWS - 1)
            return carry

        lax.fori_loop(0, tokens_per_worker // LANES, clamp_body, 0)

        def chunk_body(ci, carry):
            pltpu.async_copy(
                tab_hbm.at[idx_v.at[pl.ds(ci * chunk, chunk)]], rows_v, sem
            ).wait()
            pltpu.sync_copy(rows_v, out_hbm.at[pl.ds(base + ci * chunk, chunk)])
            return carry

        lax.fori_loop(0, n_chunks, chunk_body, 0)

    return sc_gather


def kernel(byte_ids, table, gamma, beta):
    batch, seq = byte_ids.shape
    total = batch * seq
    ids_flat = byte_ids.reshape(total).astype(jnp.int32)
    tab_n = _normalize_table(table, gamma, beta)
    out = _make_sc_gather(total)(tab_n, ids_flat)
    return out.reshape(batch, seq, D_MODEL)


# speedup vs baseline: 2.0744x; 1.0334x over previous
import functools

import jax
import jax.numpy as jnp
from jax import lax
from jax.experimental import pallas as pl
from jax.experimental.pallas import tpu as pltpu
from jax.experimental.pallas import tpu_sc as plsc

D_MODEL = 1024
NUM_ROWS = 256
NUM_CORES = 2
NUM_SUBCORES = 16
NUM_WORKERS = NUM_CORES * NUM_SUBCORES
LANES = 16


def _ln_table_body(table_ref, gamma_ref, beta_ref, out_ref):
    x = table_ref[...]
    mean = jnp.mean(x, axis=1, keepdims=True)
    cent = x - mean
    var = jnp.mean(cent * cent, axis=1, keepdims=True)
    inv = lax.rsqrt(var + 1e-5)
    out_ref[...] = cent * inv * gamma_ref[...] + beta_ref[...]


def _normalize_table(table, gamma, beta):
    return pl.pallas_call(
        _ln_table_body,
        out_shape=jax.ShapeDtypeStruct((NUM_ROWS, D_MODEL), jnp.float32),
    )(table, gamma.reshape(1, D_MODEL), beta.reshape(1, D_MODEL))


def _make_sc_gather(total_tokens):
    assert total_tokens % (8 * NUM_WORKERS) == 0
    tokens_per_worker = total_tokens // NUM_WORKERS
    chunk = 32
    n_chunks = tokens_per_worker // chunk
    mesh = plsc.VectorSubcoreMesh(
        core_axis_name="c",
        subcore_axis_name="s",
        num_cores=NUM_CORES,
        num_subcores=NUM_SUBCORES,
    )

    assert n_chunks % 2 == 0 and n_chunks >= 4

    @functools.partial(
        pl.kernel,
        out_type=jax.ShapeDtypeStruct((total_tokens, D_MODEL), jnp.float32),
        mesh=mesh,
        scratch_types=[
            pltpu.VMEM((tokens_per_worker,), jnp.int32),
            pltpu.VMEM((chunk, D_MODEL), jnp.float32),
            pltpu.VMEM((chunk, D_MODEL), jnp.float32),
            pltpu.SemaphoreType.DMA,
            pltpu.SemaphoreType.DMA,
            pltpu.SemaphoreType.DMA,
            pltpu.SemaphoreType.DMA,
        ],
    )
    def sc_gather(tab_hbm, idx_hbm, out_hbm, idx_v, buf0, buf1, g0, g1, s0, s1):
        wid = lax.axis_index("s") * NUM_CORES + lax.axis_index("c")
        base = wid * tokens_per_worker
        pltpu.sync_copy(idx_hbm.at[pl.ds(base, tokens_per_worker)], idx_v)

        def clamp_body(i, carry):
            v = idx_v[pl.ds(i * LANES, LANES)]
            idx_v[pl.ds(i * LANES, LANES)] = jnp.clip(v, 0, NUM_ROWS - 1)
            return carry

        lax.fori_loop(0, tokens_per_worker // LANES, clamp_body, 0)

        def start_gather(ci, buf, sem):
            pltpu.async_copy(tab_hbm.at[idx_v.at[pl.ds(ci * chunk, chunk)]], buf, sem)

        def start_scatter(ci, buf, sem):
            pltpu.async_copy(buf, out_hbm.at[pl.ds(base + ci * chunk, chunk)], sem)

        def wait_gather(buf, sem):
            pltpu.make_async_copy(tab_hbm.at[pl.ds(0, chunk)], buf, sem).wait()

        def wait_scatter(buf, sem):
            pltpu.make_async_copy(buf, out_hbm.at[pl.ds(base, chunk)], sem).wait()

        start_gather(0, buf0, g0)
        start_gather(1, buf1, g1)

        def pair_body(p, carry):
            ci = p * 2
            wait_gather(buf0, g0)
            start_scatter(ci, buf0, s0)
            wait_gather(buf1, g1)
            start_scatter(ci + 1, buf1, s1)
            wait_scatter(buf0, s0)
            start_gather(ci + 2, buf0, g0)
            wait_scatter(buf1, s1)
            start_gather(ci + 3, buf1, g1)
            return carry

        lax.fori_loop(0, n_chunks // 2 - 1, pair_body, 0)

        last = n_chunks - 2
        wait_gather(buf0, g0)
        start_scatter(last, buf0, s0)
        wait_gather(buf1, g1)
        start_scatter(last + 1, buf1, s1)
        wait_scatter(buf0, s0)
        wait_scatter(buf1, s1)

    return sc_gather


def kernel(byte_ids, table, gamma, beta):
    batch, seq = byte_ids.shape
    total = batch * seq
    ids_flat = byte_ids.reshape(total).astype(jnp.int32)
    tab_n = _normalize_table(table, gamma, beta)
    out = _make_sc_gather(total)(tab_n, ids_flat)
    return out.reshape(batch, seq, D_MODEL)
